# baseline (speedup 1.0000x reference)
import functools

import jax
import jax.numpy as jnp
from jax import lax
from jax.experimental import pallas as pl
from jax.experimental.pallas import tpu as pltpu
from jax.experimental.pallas import tpu_sc as plsc

N = 10000
E = 320000
D = 128
H1 = 1550
H1P = 1664
H2 = 512
C = 50
G = 64

NS = 16
CHUNK = 128
ECP = 2560
CPT = ECP // NS
NP_ = 10112
STRIPE = NP_ // NS
IBLK = 16
RB = 400
NB = N // RB

_f32 = jnp.float32
_HIGH = lax.Precision.HIGHEST
_H3 = lax.Precision.HIGH

C0_CPT = 92
C1_CPT = 68
C0_CHUNKS = C0_CPT * NS


def _sc_mesh():
  return plsc.VectorSubcoreMesh(core_axis_name="c", subcore_axis_name="s")


def _sc_layer1(x, srcf, dstf, z128, ones128):
  @functools.partial(
      pl.kernel,
      out_type=(jax.ShapeDtypeStruct((NP_, D), _f32),
                jax.ShapeDtypeStruct((NP_, D), _f32),
                jax.ShapeDtypeStruct((NP_, D), _f32)),
      mesh=_sc_mesh(),
      scratch_types=[
          [pltpu.VMEM((CHUNK,), jnp.int32)] * 2,
          [pltpu.VMEM((CHUNK,), jnp.int32)] * 2,
          [pltpu.VMEM((CHUNK, D), _f32)] * 2,
          pltpu.VMEM_SHARED((NP_, D), _f32),
          [pltpu.SemaphoreType.DMA] * 2,
          [pltpu.SemaphoreType.DMA] * 2,
          [pltpu.SemaphoreType.DMA] * 2,
      ],
  )
  def k(x_hbm, src_hbm, dst_hbm, z128_hbm, ones_hbm,
        agga_hbm, aggb_hbm, cnt_hbm,
        src_v, dst_v, rows_v, acc_sh, gsem, ssem, isem):
    cid = lax.axis_index("c")
    sid = lax.axis_index("s")
    row0 = sid * STRIPE

    def load_idx(p, c, base, with_src):
      off = pl.multiple_of((base + c) * CHUNK, CHUNK)
      dd = pltpu.async_copy(dst_hbm.at[pl.ds(off, CHUNK)], dst_v[p], isem[p])
      if with_src:
        ds_ = pltpu.async_copy(src_hbm.at[pl.ds(off, CHUNK)], src_v[p],
                               isem[p])
        ds_.wait()
      dd.wait()

    def start_gather(p):
      pltpu.async_copy(x_hbm.at[src_v[p]], rows_v[p], gsem[p])

    def wait_gather(p):
      pltpu.make_async_copy(x_hbm.at[src_v[p]], rows_v[p], gsem[p]).wait()

    def start_scatter(p):
      pltpu.async_copy(rows_v[p], acc_sh.at[dst_v[p]], ssem[p], add=True)

    def wait_scatter(p):
      pltpu.make_async_copy(rows_v[p], acc_sh.at[dst_v[p]], ssem[p]).wait()

    def zero_acc():
      pltpu.sync_copy(z128_hbm.at[pl.ds(row0, STRIPE)],
                      acc_sh.at[pl.ds(row0, STRIPE)])

    def flush(out_hbm):
      pltpu.sync_copy(acc_sh.at[pl.ds(row0, STRIPE)],
                      out_hbm.at[pl.ds(row0, STRIPE)])

    def agg_job(base, cpt, out_hbm):
      zero_acc()
      plsc.subcore_barrier()
      for p in (0, 1):
        load_idx(p, p, base, True)
        start_gather(p)

      @pl.loop(0, cpt - 2, step=2)
      def _(j):
        for p in (0, 1):
          wait_gather(p)
          start_scatter(p)
          wait_scatter(p)
          load_idx(p, j + p + 2, base, True)
          start_gather(p)

      for p in (0, 1):
        wait_gather(p)
        start_scatter(p)
        wait_scatter(p)
      plsc.subcore_barrier()
      flush(out_hbm)
      plsc.subcore_barrier()

    @pl.when(cid == 0)
    def _():
      agg_job(sid * C0_CPT, C0_CPT, agga_hbm)

    @pl.when(cid == 1)
    def _():
      agg_job(C0_CHUNKS + sid * C1_CPT, C1_CPT, aggb_hbm)

      zero_acc()
      pltpu.sync_copy(ones_hbm, rows_v[0])
      pltpu.sync_copy(ones_hbm, rows_v[1])
      plsc.subcore_barrier()
      base = sid * CPT
      for p in (0, 1):
        load_idx(p, p, base, False)
        start_scatter(p)

      @pl.loop(0, CPT - 2, step=2)
      def _(j):
        for p in (0, 1):
          wait_scatter(p)
          load_idx(p, j + p + 2, base, False)
          start_scatter(p)

      for p in (0, 1):
        wait_scatter(p)
      plsc.subcore_barrier()
      flush(cnt_hbm)

  return k(x, srcf, dstf, z128, ones128)


def _sc_layer2(pa, pb, pc, pd, srcf, dstf, z128):
  @functools.partial(
      pl.kernel,
      out_type=tuple(jax.ShapeDtypeStruct((NP_, D), _f32) for _ in range(4)),
      mesh=_sc_mesh(),
      scratch_types=[
          [pltpu.VMEM((CHUNK,), jnp.int32)] * 2,
          [pltpu.VMEM((CHUNK,), jnp.int32)] * 2,
          [pltpu.VMEM((CHUNK, D), _f32)] * 2,
          pltpu.VMEM_SHARED((NP_, D), _f32),
          [pltpu.SemaphoreType.DMA] * 2,
          [pltpu.SemaphoreType.DMA] * 2,
          [pltpu.SemaphoreType.DMA] * 2,
      ],
  )
  def k(pa_hbm, pb_hbm, pc_hbm, pd_hbm, src_hbm, dst_hbm, z_hbm,
        oa_hbm, ob_hbm, oc_hbm, od_hbm,
        src_v, dst_v, rows_v, acc_sh, gsem, ssem, isem):
    cid = lax.axis_index("c")
    sid = lax.axis_index("s")
    base = sid * CPT
    row0 = sid * STRIPE

    def load_idx(p, c):
      off = pl.multiple_of((base + c) * CHUNK, CHUNK)
      dd = pltpu.async_copy(dst_hbm.at[pl.ds(off, CHUNK)], dst_v[p], isem[p])
      ds_ = pltpu.async_copy(src_hbm.at[pl.ds(off, CHUNK)], src_v[p], isem[p])
      ds_.wait()
      dd.wait()

    def run(table_hbm, out_hbm):
      def start_gather(p):
        pltpu.async_copy(table_hbm.at[src_v[p]], rows_v[p], gsem[p])

      def wait_gather(p):
        pltpu.make_async_copy(table_hbm.at[src_v[p]], rows_v[p],
                              gsem[p]).wait()

      def start_scatter(p):
        pltpu.async_copy(rows_v[p], acc_sh.at[dst_v[p]], ssem[p], add=True)

      def wait_scatter(p):
        pltpu.make_async_copy(rows_v[p], acc_sh.at[dst_v[p]], ssem[p]).wait()

      pltpu.sync_copy(z_hbm.at[pl.ds(row0, STRIPE)],
                      acc_sh.at[pl.ds(row0, STRIPE)])
      plsc.subcore_barrier()
      for p in (0, 1):
        load_idx(p, p)
        start_gather(p)

      @pl.loop(0, CPT - 2, step=2)
      def _(j):
        for p in (0, 1):
          wait_gather(p)
          start_scatter(p)
          wait_scatter(p)
          load_idx(p, j + p + 2)
          start_gather(p)

      for p in (0, 1):
        wait_gather(p)
        start_scatter(p)
        wait_scatter(p)
      plsc.subcore_barrier()
      pltpu.sync_copy(acc_sh.at[pl.ds(row0, STRIPE)],
                      out_hbm.at[pl.ds(row0, STRIPE)])
      plsc.subcore_barrier()

    @pl.when(cid == 0)
    def _():
      run(pa_hbm, oa_hbm)
      run(pb_hbm, ob_hbm)

    @pl.when(cid == 1)
    def _():
      run(pc_hbm, oc_hbm)
      run(pd_hbm, od_hbm)

  return k(pa, pb, pc, pd, srcf, dstf, z128)


def _tc_dense1(x, agg1a, agg1b, cnt, w1l, w1r, c1, w2l, w2r):
  def body(x_ref, a_ref, b_ref, c_ref, w1l_ref, w1r_ref, c1_ref, w2l_ref,
           w2r_ref, pa_ref, pb_ref, pc_ref, pd_ref, r2_ref):
    deg = jnp.clip(c_ref[:, 0:1], 1.0, None)
    mean = (a_ref[...] + b_ref[...]) / deg
    h = jnp.dot(mean, w1l_ref[...], preferred_element_type=_f32,
                precision=_HIGH)
    h = h + jnp.dot(x_ref[...], w1r_ref[...], preferred_element_type=_f32,
                    precision=_HIGH)
    h = jnp.maximum(h + c1_ref[...], 0.0)
    p2 = jnp.dot(h, w2l_ref[...], preferred_element_type=_f32,
                 precision=_HIGH)
    r2 = jnp.dot(h, w2r_ref[...], preferred_element_type=_f32,
                 precision=_HIGH)
    pa_ref[...] = p2[:, 0:128]
    pb_ref[...] = p2[:, 128:256]
    pc_ref[...] = p2[:, 256:384]
    pd_ref[...] = p2[:, 384:512]
    r2_ref[...] = r2

  n128 = jax.ShapeDtypeStruct((N, 128), _f32)
  return pl.pallas_call(
      body,
      grid=(NB,),
      in_specs=[
          pl.BlockSpec((RB, D), lambda i: (i, 0)),
          pl.BlockSpec((RB, D), lambda i: (i, 0)),
          pl.BlockSpec((RB, D), lambda i: (i, 0)),
          pl.BlockSpec((RB, 128), lambda i: (i, 0)),
          pl.BlockSpec((D, H1P), lambda i: (0, 0)),
          pl.BlockSpec((D, H1P), lambda i: (0, 0)),
          pl.BlockSpec((1, H1P), lambda i: (0, 0)),
          pl.BlockSpec((H1P, H2), lambda i: (0, 0)),
          pl.BlockSpec((H1P, H2), lambda i: (0, 0)),
      ],
      out_specs=[
          pl.BlockSpec((RB, 128), lambda i: (i, 0)),
          pl.BlockSpec((RB, 128), lambda i: (i, 0)),
          pl.BlockSpec((RB, 128), lambda i: (i, 0)),
          pl.BlockSpec((RB, 128), lambda i: (i, 0)),
          pl.BlockSpec((RB, H2), lambda i: (i, 0)),
      ],
      out_shape=(n128, n128, n128, n128, jax.ShapeDtypeStruct((N, H2), _f32)),
  )(x, agg1a, agg1b, cnt, w1l, w1r, c1, w2l, w2r)


def _tc_dense2(aa, ab, ac, ad, cnt, r2, c2, batch3, wf, bf):
  def body(aa_ref, ab_ref, ac_ref, ad_ref, c_ref, r2_ref, c2_ref, b3_ref,
           wf_ref, bf_ref, out_ref, pooled, gcnt):
    i = pl.program_id(0)

    @pl.when(i == 0)
    def _():
      pooled[...] = jnp.zeros_like(pooled)
      gcnt[...] = jnp.zeros_like(gcnt)

    inv = 1.0 / jnp.clip(c_ref[:, 0:1], 1.0, None)
    parts = []
    for c, aref in enumerate((aa_ref, ab_ref, ac_ref, ad_ref)):
      m = aref[...] * inv
      parts.append(m + r2_ref[:, c * 128:(c + 1) * 128]
                   + c2_ref[:, c * 128:(c + 1) * 128])
    h2 = jnp.maximum(jnp.concatenate(parts, axis=1), 0.0)

    b = b3_ref[0, 0, :]
    onehot_t = (b[None, :] == lax.broadcasted_iota(jnp.int32, (G, RB), 0)
                ).astype(_f32)
    gcnt[...] = gcnt[...] + jnp.sum(onehot_t, axis=1, keepdims=True)
    pooled[...] = pooled[...] + lax.dot_general(
        onehot_t, h2, (((1,), (0,)), ((), ())),
        preferred_element_type=_f32, precision=_HIGH)

    @pl.when(i == NB - 1)
    def _():
      cc = jnp.clip(gcnt[:, 0:1], 1.0, None)
      pm = pooled[...] / cc
      logits = jnp.dot(pm, wf_ref[...], preferred_element_type=_f32,
                       precision=_HIGH) + bf_ref[...]
      col = lax.broadcasted_iota(jnp.int32, (G, 128), 1)
      lm = jnp.where(col < C, logits, jnp.float32(-1e30))
      mx = jnp.max(lm, axis=1, keepdims=True)
      ex = jnp.where(col < C, jnp.exp(lm - mx), 0.0)
      lse = jnp.log(jnp.sum(ex, axis=1, keepdims=True)) + mx
      out_ref[...] = (lm - lse)[:, :C]

  return pl.pallas_call(
      body,
      grid=(NB,),
      in_specs=[
          pl.BlockSpec((RB, 128), lambda i: (i, 0)),
          pl.BlockSpec((RB, 128), lambda i: (i, 0)),
          pl.BlockSpec((RB, 128), lambda i: (i, 0)),
          pl.BlockSpec((RB, 128), lambda i: (i, 0)),
          pl.BlockSpec((RB, 128), lambda i: (i, 0)),
          pl.BlockSpec((RB, H2), lambda i: (i, 0)),
          pl.BlockSpec((1, H2), lambda i: (0, 0)),
          pl.BlockSpec((1, 1, RB), lambda i: (i, 0, 0)),
          pl.BlockSpec((H2, 128), lambda i: (0, 0)),
          pl.BlockSpec((1, 128), lambda i: (0, 0)),
      ],
      out_specs=pl.BlockSpec((G, C), lambda i: (0, 0)),
      out_shape=jax.ShapeDtypeStruct((G, C), _f32),
      scratch_shapes=[
          pltpu.VMEM((G, H2), _f32),
          pltpu.VMEM((G, 128), _f32),
      ],
  )(aa, ab, ac, ad, cnt, r2, c2, batch3, wf, bf)


def kernel(x, edge_index, batch, W1l, b1l, W1r, bn1_g, bn1_b, bn1_m, bn1_v,
           W2l, b2l, W2r, bn2_g, bn2_b, bn2_m, bn2_v, Wf, bf):
  eps = 1e-5
  s1 = bn1_g / jnp.sqrt(bn1_v + eps)
  t1 = bn1_b - bn1_m * s1
  w1l = jnp.pad(W1l * s1[None, :], ((0, 0), (0, H1P - H1)))
  w1r = jnp.pad(W1r * s1[None, :], ((0, 0), (0, H1P - H1)))
  c1 = jnp.pad(b1l * s1 + t1, (0, H1P - H1))[None, :]
  s2 = bn2_g / jnp.sqrt(bn2_v + eps)
  t2 = bn2_b - bn2_m * s2
  w2l = jnp.pad(W2l * s2[None, :], ((0, H1P - H1), (0, 0)))
  w2r = jnp.pad(W2r * s2[None, :], ((0, H1P - H1), (0, 0)))
  c2 = (b2l * s2 + t2)[None, :]
  wf = jnp.pad(Wf, ((0, 0), (0, 128 - C)))
  bfp = jnp.pad(bf, (0, 128 - C))[None, :]

  src = edge_index[0].astype(jnp.int32)
  dst = edge_index[1].astype(jnp.int32)
  srcf = jnp.pad(src, (0, ECP * CHUNK - E))
  dstf = jnp.pad(dst, (0, ECP * CHUNK - E), constant_values=N)
  z128 = jnp.zeros((NP_, D), _f32)
  ones128 = jnp.ones((CHUNK, D), _f32)
  batch3 = batch.astype(jnp.int32).reshape(NB, 1, RB)
  xf = x.astype(_f32)

  agg1ap, agg1bp, cntp = _sc_layer1(xf, srcf, dstf, z128, ones128)
  cnt = cntp[:N]
  pa, pb, pc, pd, r2 = _tc_dense1(xf, agg1ap[:N], agg1bp[:N], cnt,
                                  w1l, w1r, c1, w2l, w2r)
  oa, ob, oc, od = _sc_layer2(pa, pb, pc, pd, srcf, dstf, z128)
  return _tc_dense2(oa[:N], ob[:N], oc[:N], od[:N], cnt, r2, c2, batch3,
                    wf, bfp)


# speedup vs baseline: 1.1301x; 1.1301x over previous
import functools

import jax
import jax.numpy as jnp
from jax import lax
from jax.experimental import pallas as pl
from jax.experimental.pallas import tpu as pltpu
from jax.experimental.pallas import tpu_sc as plsc

N = 10000
E = 320000
D = 128
H1 = 1550
H1P = 1664
H2 = 512
C = 50
G = 64

NS = 16
CHUNK = 128
ECP = 2560
CPT = ECP // NS
NP_ = 10112
STRIPE = NP_ // NS
IBLK = 16
RB = 400
NB = N // RB

_f32 = jnp.float32
_HIGH = lax.Precision.HIGHEST


def _sc_mesh():
  return plsc.VectorSubcoreMesh(core_axis_name="c", subcore_axis_name="s")


def _sc_layer1(x, srcf, dstf, z128, ones128):
  @functools.partial(
      pl.kernel,
      out_type=(jax.ShapeDtypeStruct((NP_, D), _f32),
                jax.ShapeDtypeStruct((NP_, D), _f32)),
      mesh=_sc_mesh(),
      scratch_types=[
          [pltpu.VMEM((CHUNK,), jnp.int32)] * 2,
          [pltpu.VMEM((CHUNK,), jnp.int32)] * 2,
          [pltpu.VMEM((CHUNK, D), _f32)] * 2,
          pltpu.VMEM_SHARED((NP_, D), _f32),
          [pltpu.SemaphoreType.DMA] * 2,
          [pltpu.SemaphoreType.DMA] * 2,
          [pltpu.SemaphoreType.DMA] * 2,
      ],
  )
  def k(x_hbm, src_hbm, dst_hbm, z128_hbm, ones_hbm,
        agg_hbm, cnt_hbm,
        src_v, dst_v, rows_v, acc_sh, gsem, ssem, isem):
    cid = lax.axis_index("c")
    sid = lax.axis_index("s")
    base = sid * CPT
    row0 = sid * STRIPE

    def load_idx(p, c, with_src):
      off = pl.multiple_of((base + c) * CHUNK, CHUNK)
      dd = pltpu.async_copy(dst_hbm.at[pl.ds(off, CHUNK)], dst_v[p], isem[p])
      if with_src:
        ds_ = pltpu.async_copy(src_hbm.at[pl.ds(off, CHUNK)], src_v[p],
                               isem[p])
        ds_.wait()
      dd.wait()

    def start_gather(p):
      pltpu.async_copy(x_hbm.at[src_v[p]], rows_v[p], gsem[p])

    def wait_gather(p):
      pltpu.make_async_copy(x_hbm.at[src_v[p]], rows_v[p], gsem[p]).wait()

    def start_scatter(p):
      pltpu.async_copy(rows_v[p], acc_sh.at[dst_v[p]], ssem[p], add=True)

    def wait_scatter(p):
      pltpu.make_async_copy(rows_v[p], acc_sh.at[dst_v[p]], ssem[p]).wait()

    pltpu.sync_copy(z128_hbm.at[pl.ds(row0, STRIPE)],
                    acc_sh.at[pl.ds(row0, STRIPE)])

    @pl.when(cid == 0)
    def _():
      plsc.subcore_barrier()
      for p in (0, 1):
        load_idx(p, p, True)
        start_gather(p)

      @pl.loop(0, CPT - 2, step=2)
      def _(j):
        for p in (0, 1):
          wait_gather(p)
          start_scatter(p)
          wait_scatter(p)
          load_idx(p, j + p + 2, True)
          start_gather(p)

      for p in (0, 1):
        wait_gather(p)
        start_scatter(p)
        wait_scatter(p)
      plsc.subcore_barrier()
      pltpu.sync_copy(acc_sh.at[pl.ds(row0, STRIPE)],
                      agg_hbm.at[pl.ds(row0, STRIPE)])

    @pl.when(cid == 1)
    def _():
      pltpu.sync_copy(ones_hbm, rows_v[0])
      pltpu.sync_copy(ones_hbm, rows_v[1])
      plsc.subcore_barrier()
      for p in (0, 1):
        load_idx(p, p, False)
        start_scatter(p)

      @pl.loop(0, CPT - 2, step=2)
      def _(j):
        for p in (0, 1):
          wait_scatter(p)
          load_idx(p, j + p + 2, False)
          start_scatter(p)

      for p in (0, 1):
        wait_scatter(p)
      plsc.subcore_barrier()
      pltpu.sync_copy(acc_sh.at[pl.ds(row0, STRIPE)],
                      cnt_hbm.at[pl.ds(row0, STRIPE)])

  return k(x, srcf, dstf, z128, ones128)


def _sc_layer2(pa, pb, pc, pd, srcf, dstf, z128):
  @functools.partial(
      pl.kernel,
      out_type=tuple(jax.ShapeDtypeStruct((NP_, D), _f32) for _ in range(4)),
      mesh=_sc_mesh(),
      scratch_types=[
          [pltpu.VMEM((CHUNK,), jnp.int32)] * 2,
          [pltpu.VMEM((CHUNK,), jnp.int32)] * 2,
          [pltpu.VMEM((CHUNK, D), _f32)] * 2,
          pltpu.VMEM_SHARED((NP_, D), _f32),
          [pltpu.SemaphoreType.DMA] * 2,
          [pltpu.SemaphoreType.DMA] * 2,
          [pltpu.SemaphoreType.DMA] * 2,
      ],
  )
  def k(pa_hbm, pb_hbm, pc_hbm, pd_hbm, src_hbm, dst_hbm, z_hbm,
        oa_hbm, ob_hbm, oc_hbm, od_hbm,
        src_v, dst_v, rows_v, acc_sh, gsem, ssem, isem):
    cid = lax.axis_index("c")
    sid = lax.axis_index("s")
    base = sid * CPT
    row0 = sid * STRIPE

    def load_idx(p, c):
      off = pl.multiple_of((base + c) * CHUNK, CHUNK)
      dd = pltpu.async_copy(dst_hbm.at[pl.ds(off, CHUNK)], dst_v[p], isem[p])
      ds_ = pltpu.async_copy(src_hbm.at[pl.ds(off, CHUNK)], src_v[p], isem[p])
      ds_.wait()
      dd.wait()

    def run(table_hbm, out_hbm):
      def start_gather(p):
        pltpu.async_copy(table_hbm.at[src_v[p]], rows_v[p], gsem[p])

      def wait_gather(p):
        pltpu.make_async_copy(table_hbm.at[src_v[p]], rows_v[p],
                              gsem[p]).wait()

      def start_scatter(p):
        pltpu.async_copy(rows_v[p], acc_sh.at[dst_v[p]], ssem[p], add=True)

      def wait_scatter(p):
        pltpu.make_async_copy(rows_v[p], acc_sh.at[dst_v[p]], ssem[p]).wait()

      pltpu.sync_copy(z_hbm.at[pl.ds(row0, STRIPE)],
                      acc_sh.at[pl.ds(row0, STRIPE)])
      plsc.subcore_barrier()
      for p in (0, 1):
        load_idx(p, p)
        start_gather(p)

      @pl.loop(0, CPT - 2, step=2)
      def _(j):
        for p in (0, 1):
          wait_gather(p)
          start_scatter(p)
          wait_scatter(p)
          load_idx(p, j + p + 2)
          start_gather(p)

      for p in (0, 1):
        wait_gather(p)
        start_scatter(p)
        wait_scatter(p)
      plsc.subcore_barrier()
      pltpu.sync_copy(acc_sh.at[pl.ds(row0, STRIPE)],
                      out_hbm.at[pl.ds(row0, STRIPE)])
      plsc.subcore_barrier()

    @pl.when(cid == 0)
    def _():
      run(pa_hbm, oa_hbm)
      run(pb_hbm, ob_hbm)

    @pl.when(cid == 1)
    def _():
      run(pc_hbm, oc_hbm)
      run(pd_hbm, od_hbm)

  return k(pa, pb, pc, pd, srcf, dstf, z128)


def _split_bf16(w):
  hi = w.astype(jnp.bfloat16)
  lo = (w - hi.astype(_f32)).astype(jnp.bfloat16)
  return hi, lo


def _dot3(a, w_hi, w_lo):
  a_hi = a.astype(jnp.bfloat16)
  a_lo = (a - a_hi.astype(_f32)).astype(jnp.bfloat16)
  out = jnp.dot(a_hi, w_hi, preferred_element_type=_f32)
  out += jnp.dot(a_hi, w_lo, preferred_element_type=_f32)
  out += jnp.dot(a_lo, w_hi, preferred_element_type=_f32)
  return out


def _tc_dense1(x, agg1, cnt, w1l_hi, w1l_lo, w1r_hi, w1r_lo, c1,
               w2l_hi, w2l_lo, w2r_hi, w2r_lo):
  def body(x_ref, a_ref, c_ref, w1lh_ref, w1ll_ref, w1rh_ref, w1rl_ref,
           c1_ref, w2lh_ref, w2ll_ref, w2rh_ref, w2rl_ref,
           pa_ref, pb_ref, pc_ref, pd_ref, r2_ref):
    deg = jnp.clip(c_ref[:, 0:1], 1.0, None)
    mean = a_ref[...] / deg
    h = _dot3(mean, w1lh_ref[...], w1ll_ref[...])
    h = h + _dot3(x_ref[...], w1rh_ref[...], w1rl_ref[...])
    h = jnp.maximum(h + c1_ref[...], 0.0)
    p2 = _dot3(h, w2lh_ref[...], w2ll_ref[...])
    r2 = _dot3(h, w2rh_ref[...], w2rl_ref[...])
    pa_ref[...] = p2[:, 0:128]
    pb_ref[...] = p2[:, 128:256]
    pc_ref[...] = p2[:, 256:384]
    pd_ref[...] = p2[:, 384:512]
    r2_ref[...] = r2

  n128 = jax.ShapeDtypeStruct((N, 128), _f32)
  wspec1 = pl.BlockSpec((D, H1P), lambda i: (0, 0))
  wspec2 = pl.BlockSpec((H1P, H2), lambda i: (0, 0))
  return pl.pallas_call(
      body,
      grid=(NB,),
      in_specs=[
          pl.BlockSpec((RB, D), lambda i: (i, 0)),
          pl.BlockSpec((RB, D), lambda i: (i, 0)),
          pl.BlockSpec((RB, 128), lambda i: (i, 0)),
          wspec1,
          wspec1,
          wspec1,
          wspec1,
          pl.BlockSpec((1, H1P), lambda i: (0, 0)),
          wspec2,
          wspec2,
          wspec2,
          wspec2,
      ],
      out_specs=[
          pl.BlockSpec((RB, 128), lambda i: (i, 0)),
          pl.BlockSpec((RB, 128), lambda i: (i, 0)),
          pl.BlockSpec((RB, 128), lambda i: (i, 0)),
          pl.BlockSpec((RB, 128), lambda i: (i, 0)),
          pl.BlockSpec((RB, H2), lambda i: (i, 0)),
      ],
      out_shape=(n128, n128, n128, n128, jax.ShapeDtypeStruct((N, H2), _f32)),
  )(x, agg1, cnt, w1l_hi, w1l_lo, w1r_hi, w1r_lo, c1,
    w2l_hi, w2l_lo, w2r_hi, w2r_lo)


def _tc_dense2(aa, ab, ac, ad, cnt, r2, c2, batch3, wf, bf):
  def body(aa_ref, ab_ref, ac_ref, ad_ref, c_ref, r2_ref, c2_ref, b3_ref,
           wf_ref, bf_ref, out_ref, pooled, gcnt):
    i = pl.program_id(0)

    @pl.when(i == 0)
    def _():
      pooled[...] = jnp.zeros_like(pooled)
      gcnt[...] = jnp.zeros_like(gcnt)

    inv = 1.0 / jnp.clip(c_ref[:, 0:1], 1.0, None)
    parts = []
    for c, aref in enumerate((aa_ref, ab_ref, ac_ref, ad_ref)):
      m = aref[...] * inv
      parts.append(m + r2_ref[:, c * 128:(c + 1) * 128]
                   + c2_ref[:, c * 128:(c + 1) * 128])
    h2 = jnp.maximum(jnp.concatenate(parts, axis=1), 0.0)

    b = b3_ref[0, 0, :]
    onehot_t = (b[None, :] == lax.broadcasted_iota(jnp.int32, (G, RB), 0)
                ).astype(_f32)
    gcnt[...] = gcnt[...] + jnp.sum(onehot_t, axis=1, keepdims=True)
    pooled[...] = pooled[...] + lax.dot_general(
        onehot_t, h2, (((1,), (0,)), ((), ())),
        preferred_element_type=_f32, precision=_HIGH)

    @pl.when(i == NB - 1)
    def _():
      cc = jnp.clip(gcnt[:, 0:1], 1.0, None)
      pm = pooled[...] / cc
      logits = jnp.dot(pm, wf_ref[...], preferred_element_type=_f32,
                       precision=_HIGH) + bf_ref[...]
      col = lax.broadcasted_iota(jnp.int32, (G, 128), 1)
      lm = jnp.where(col < C, logits, jnp.float32(-1e30))
      mx = jnp.max(lm, axis=1, keepdims=True)
      ex = jnp.where(col < C, jnp.exp(lm - mx), 0.0)
      lse = jnp.log(jnp.sum(ex, axis=1, keepdims=True)) + mx
      out_ref[...] = (lm - lse)[:, :C]

  return pl.pallas_call(
      body,
      grid=(NB,),
      in_specs=[
          pl.BlockSpec((RB, 128), lambda i: (i, 0)),
          pl.BlockSpec((RB, 128), lambda i: (i, 0)),
          pl.BlockSpec((RB, 128), lambda i: (i, 0)),
          pl.BlockSpec((RB, 128), lambda i: (i, 0)),
          pl.BlockSpec((RB, 128), lambda i: (i, 0)),
          pl.BlockSpec((RB, H2), lambda i: (i, 0)),
          pl.BlockSpec((1, H2), lambda i: (0, 0)),
          pl.BlockSpec((1, 1, RB), lambda i: (i, 0, 0)),
          pl.BlockSpec((H2, 128), lambda i: (0, 0)),
          pl.BlockSpec((1, 128), lambda i: (0, 0)),
      ],
      out_specs=pl.BlockSpec((G, C), lambda i: (0, 0)),
      out_shape=jax.ShapeDtypeStruct((G, C), _f32),
      scratch_shapes=[
          pltpu.VMEM((G, H2), _f32),
          pltpu.VMEM((G, 128), _f32),
      ],
  )(aa, ab, ac, ad, cnt, r2, c2, batch3, wf, bf)


def kernel(x, edge_index, batch, W1l, b1l, W1r, bn1_g, bn1_b, bn1_m, bn1_v,
           W2l, b2l, W2r, bn2_g, bn2_b, bn2_m, bn2_v, Wf, bf):
  eps = 1e-5
  s1 = bn1_g / jnp.sqrt(bn1_v + eps)
  t1 = bn1_b - bn1_m * s1
  w1l = jnp.pad(W1l * s1[None, :], ((0, 0), (0, H1P - H1)))
  w1r = jnp.pad(W1r * s1[None, :], ((0, 0), (0, H1P - H1)))
  c1 = jnp.pad(b1l * s1 + t1, (0, H1P - H1))[None, :]
  s2 = bn2_g / jnp.sqrt(bn2_v + eps)
  t2 = bn2_b - bn2_m * s2
  w2l = jnp.pad(W2l * s2[None, :], ((0, H1P - H1), (0, 0)))
  w2r = jnp.pad(W2r * s2[None, :], ((0, H1P - H1), (0, 0)))
  c2 = (b2l * s2 + t2)[None, :]
  wf = jnp.pad(Wf, ((0, 0), (0, 128 - C)))
  bfp = jnp.pad(bf, (0, 128 - C))[None, :]

  src = edge_index[0].astype(jnp.int32)
  dst = edge_index[1].astype(jnp.int32)
  srcf = jnp.pad(src, (0, ECP * CHUNK - E))
  dstf = jnp.pad(dst, (0, ECP * CHUNK - E), constant_values=N)
  z128 = jnp.zeros((NP_, D), _f32)
  ones128 = jnp.ones((CHUNK, D), _f32)
  batch3 = batch.astype(jnp.int32).reshape(NB, 1, RB)
  xf = x.astype(_f32)

  agg1p, cntp = _sc_layer1(xf, srcf, dstf, z128, ones128)
  agg1 = agg1p[:N]
  cnt = cntp[:N]
  w1l_hi, w1l_lo = _split_bf16(w1l)
  w1r_hi, w1r_lo = _split_bf16(w1r)
  w2l_hi, w2l_lo = _split_bf16(w2l)
  w2r_hi, w2r_lo = _split_bf16(w2r)
  pa, pb, pc, pd, r2 = _tc_dense1(xf, agg1, cnt, w1l_hi, w1l_lo,
                                  w1r_hi, w1r_lo, c1,
                                  w2l_hi, w2l_lo, w2r_hi, w2r_lo)
  oa, ob, oc, od = _sc_layer2(pa, pb, pc, pd, srcf, dstf, z128)
  return _tc_dense2(oa[:N], ob[:N], oc[:N], od[:N], cnt, r2, c2, batch3,
                    wf, bfp)
